# Initial kernel scaffold; baseline (speedup 1.0000x reference)
import jax
import jax.numpy as jnp
from jax.experimental import pallas as pl


def kernel(x, edge_index, edge_weight, W1, b1, W2, b2, Wlin, blin):
    raise NotImplementedError("write your pallas kernel here")


# speedup vs baseline: 15.1495x; 15.1495x over previous
import functools

import jax
import jax.numpy as jnp
from jax import lax
from jax.experimental import pallas as pl
from jax.experimental.pallas import tpu as pltpu
from jax.experimental.pallas import tpu_sc as plsc

_N = 676
_E = 8192
_STRIDE = 704
_FLAT = _N * _STRIDE
_CHUNK = _FLAT // 16
_EPW = _E // 32
_DEGP = _STRIDE


def _sc_body(src_hbm, dst_hbm, w_hbm, zeros_hbm, m_out, deg_out,
             srcv, dstv, wv, midx, zbuf, m_sp, deg_sp):
    c = lax.axis_index("c")
    s = lax.axis_index("s")
    wid = c * 16 + s
    base = wid * _EPW

    pltpu.sync_copy(zeros_hbm.at[pl.ds(0, _CHUNK)], zbuf)
    pltpu.sync_copy(zbuf, m_sp.at[pl.ds(s * _CHUNK, _CHUNK)])

    @pl.when(s == 0)
    def _():
        pltpu.sync_copy(zbuf.at[pl.ds(0, _DEGP)], deg_sp)

    for j in range(2):
        off = base + j * 128
        pltpu.sync_copy(src_hbm.at[pl.ds(off, 128)], srcv.at[j])
        pltpu.sync_copy(dst_hbm.at[pl.ds(off, 128)], dstv.at[j])
        pltpu.sync_copy(w_hbm.at[pl.ds(off, 128)], wv.at[j])

    for j in range(2):
        for k in range(8):
            sl = pl.ds(k * 16, 16)
            midx[j, sl] = srcv[j, sl] * _STRIDE + dstv[j, sl]

    plsc.subcore_barrier()

    for j in range(2):
        pltpu.sync_copy(wv.at[j], m_sp.at[midx.at[j]], add=True)
        pltpu.sync_copy(wv.at[j], deg_sp.at[dstv.at[j]], add=True)

    plsc.subcore_barrier()

    pltpu.sync_copy(m_sp.at[pl.ds(s * _CHUNK, _CHUNK)], zbuf)
    pltpu.sync_copy(zbuf, m_out.at[pl.ds(c * _FLAT + s * _CHUNK, _CHUNK)])

    @pl.when(s == 0)
    def _():
        pltpu.sync_copy(deg_sp, zbuf.at[pl.ds(0, _DEGP)])
        pltpu.sync_copy(zbuf.at[pl.ds(0, _DEGP)],
                        deg_out.at[pl.ds(c * _DEGP, _DEGP)])


@functools.lru_cache(maxsize=1)
def _sc_build_adj():
    return pl.kernel(
        _sc_body,
        out_type=[
            jax.ShapeDtypeStruct((2 * _FLAT,), jnp.float32),
            jax.ShapeDtypeStruct((2 * _DEGP,), jnp.float32),
        ],
        mesh=plsc.VectorSubcoreMesh(core_axis_name="c", subcore_axis_name="s"),
        scratch_types=[
            pltpu.VMEM((2, 128), jnp.int32),
            pltpu.VMEM((2, 128), jnp.int32),
            pltpu.VMEM((2, 128), jnp.float32),
            pltpu.VMEM((2, 128), jnp.int32),
            pltpu.VMEM((_CHUNK,), jnp.float32),
            pltpu.VMEM_SHARED((_FLAT,), jnp.float32),
            pltpu.VMEM_SHARED((_DEGP,), jnp.float32),
        ],
    )


def _tc_body(x_ref, m_ref, deg_ref, w1_ref, b1_ref, w2_ref, b2_ref,
             wlin_ref, blin_ref, out_ref, ms_ref):
    @pl.when(pl.program_id(0) == 0)
    def _():
        deg = deg_ref[0, :_N] + deg_ref[1, :_N] + 1.0
        dinv = lax.rsqrt(deg)
        mraw = m_ref[0, :, :_N] + m_ref[1, :, :_N]
        rows = lax.broadcasted_iota(jnp.int32, (_N, _N), 0)
        cols = lax.broadcasted_iota(jnp.int32, (_N, _N), 1)
        eye = jnp.where(rows == cols, 1.0, 0.0).astype(jnp.float32)
        ms_ref[...] = dinv[:, None] * (mraw + eye) * dinv[None, :]

    w1 = w1_ref[0, 0]
    b1 = b1_ref[0, 0]
    w2 = w2_ref[0, 0]
    b2 = b2_ref[0, 0]
    blin = blin_ref[0, 0]
    ms = ms_ref[...]
    h = x_ref[...]
    h = jnp.maximum(jnp.dot(h, ms, preferred_element_type=jnp.float32) * w1
                    + b1, 0.0)
    h = jnp.maximum(jnp.dot(h, ms, preferred_element_type=jnp.float32) * w2
                    + b2, 0.0)
    o = jnp.dot(h, wlin_ref[...], preferred_element_type=jnp.float32)
    out_ref[...] = jnp.maximum(o + blin, 0.0)


def kernel(x, edge_index, edge_weight, W1, b1, W2, b2, Wlin, blin):
    B = x.shape[0]
    zeros = jnp.zeros((_FLAT,), jnp.float32)
    m_parts, deg_parts = _sc_build_adj()(
        edge_index[0], edge_index[1], edge_weight, zeros)
    m_parts = m_parts.reshape(2, _N, _STRIDE)
    deg_parts = deg_parts.reshape(2, _DEGP)

    bm = 512
    grid = (B // bm,)
    out = pl.pallas_call(
        _tc_body,
        grid=grid,
        in_specs=[
            pl.BlockSpec((bm, _N), lambda i: (i, 0)),
            pl.BlockSpec((2, _N, _STRIDE), lambda i: (0, 0, 0)),
            pl.BlockSpec((2, _DEGP), lambda i: (0, 0)),
            pl.BlockSpec(memory_space=pltpu.SMEM),
            pl.BlockSpec(memory_space=pltpu.SMEM),
            pl.BlockSpec(memory_space=pltpu.SMEM),
            pl.BlockSpec(memory_space=pltpu.SMEM),
            pl.BlockSpec((_N, 1), lambda i: (0, 0)),
            pl.BlockSpec(memory_space=pltpu.SMEM),
        ],
        out_specs=pl.BlockSpec((bm, 1), lambda i: (i, 0)),
        out_shape=jax.ShapeDtypeStruct((B, 1), jnp.float32),
        scratch_shapes=[pltpu.VMEM((_N, _N), jnp.float32)],
    )(
        x, m_parts, deg_parts,
        W1.reshape(1, 1), b1.reshape(1, 1),
        W2.reshape(1, 1), b2.reshape(1, 1),
        Wlin, blin.reshape(1, 1),
    )
    return out


# speedup vs baseline: 15.2220x; 1.0048x over previous
import functools

import jax
import jax.numpy as jnp
from jax import lax
from jax.experimental import pallas as pl
from jax.experimental.pallas import tpu as pltpu
from jax.experimental.pallas import tpu_sc as plsc

_N = 676
_E = 8192
_STRIDE = 704
_FLAT = _N * _STRIDE
_CHUNK = _FLAT // 16
_EPW = _E // 32
_DEGP = _STRIDE


def _sc_body(src_hbm, dst_hbm, w_hbm, zeros_hbm, m_out, deg_out,
             srcv, dstv, wv, midx, zbuf, m_sp, deg_sp):
    c = lax.axis_index("c")
    s = lax.axis_index("s")
    wid = c * 16 + s
    base = wid * _EPW

    pltpu.sync_copy(zeros_hbm.at[pl.ds(0, _CHUNK)], zbuf)
    pltpu.sync_copy(zbuf, m_sp.at[pl.ds(s * _CHUNK, _CHUNK)])

    @pl.when(s == 0)
    def _():
        pltpu.sync_copy(zbuf.at[pl.ds(0, _DEGP)], deg_sp)

    for j in range(2):
        off = base + j * 128
        pltpu.sync_copy(src_hbm.at[pl.ds(off, 128)], srcv.at[j])
        pltpu.sync_copy(dst_hbm.at[pl.ds(off, 128)], dstv.at[j])
        pltpu.sync_copy(w_hbm.at[pl.ds(off, 128)], wv.at[j])

    for j in range(2):
        for k in range(8):
            sl = pl.ds(k * 16, 16)
            midx[j, sl] = srcv[j, sl] * _STRIDE + dstv[j, sl]

    plsc.subcore_barrier()

    for j in range(2):
        pltpu.sync_copy(wv.at[j], m_sp.at[midx.at[j]], add=True)
        pltpu.sync_copy(wv.at[j], deg_sp.at[dstv.at[j]], add=True)

    plsc.subcore_barrier()

    pltpu.sync_copy(m_sp.at[pl.ds(s * _CHUNK, _CHUNK)], zbuf)
    pltpu.sync_copy(zbuf, m_out.at[pl.ds(c * _FLAT + s * _CHUNK, _CHUNK)])

    @pl.when(s == 0)
    def _():
        pltpu.sync_copy(deg_sp, zbuf.at[pl.ds(0, _DEGP)])
        pltpu.sync_copy(zbuf.at[pl.ds(0, _DEGP)],
                        deg_out.at[pl.ds(c * _DEGP, _DEGP)])


@functools.lru_cache(maxsize=1)
def _sc_build_adj():
    return pl.kernel(
        _sc_body,
        out_type=[
            jax.ShapeDtypeStruct((2 * _FLAT,), jnp.float32),
            jax.ShapeDtypeStruct((2 * _DEGP,), jnp.float32),
        ],
        mesh=plsc.VectorSubcoreMesh(core_axis_name="c", subcore_axis_name="s"),
        scratch_types=[
            pltpu.VMEM((2, 128), jnp.int32),
            pltpu.VMEM((2, 128), jnp.int32),
            pltpu.VMEM((2, 128), jnp.float32),
            pltpu.VMEM((2, 128), jnp.int32),
            pltpu.VMEM((_CHUNK,), jnp.float32),
            pltpu.VMEM_SHARED((_FLAT,), jnp.float32),
            pltpu.VMEM_SHARED((_DEGP,), jnp.float32),
        ],
    )


def _tc_body(x_ref, m_ref, deg_ref, w1_ref, b1_ref, w2_ref, b2_ref,
             wlin_ref, blin_ref, out_ref, ms_ref):
    @pl.when(pl.program_id(0) == 0)
    def _():
        deg = deg_ref[0, :_N] + deg_ref[1, :_N] + 1.0
        dinv = lax.rsqrt(deg)
        mraw = m_ref[0, :, :_N] + m_ref[1, :, :_N]
        rows = lax.broadcasted_iota(jnp.int32, (_N, _N), 0)
        cols = lax.broadcasted_iota(jnp.int32, (_N, _N), 1)
        eye = jnp.where(rows == cols, 1.0, 0.0).astype(jnp.float32)
        ms = dinv[:, None] * (mraw + eye) * dinv[None, :]
        ms_ref[...] = ms.astype(jnp.bfloat16)

    w1 = w1_ref[0, 0]
    b1 = b1_ref[0, 0]
    w2 = w2_ref[0, 0]
    b2 = b2_ref[0, 0]
    blin = blin_ref[0, 0]
    ms = ms_ref[...]
    h = x_ref[...].astype(jnp.bfloat16)
    h = jnp.maximum(jnp.dot(h, ms, preferred_element_type=jnp.float32) * w1
                    + b1, 0.0).astype(jnp.bfloat16)
    h = jnp.maximum(jnp.dot(h, ms, preferred_element_type=jnp.float32) * w2
                    + b2, 0.0)
    o = jnp.dot(h, wlin_ref[...], preferred_element_type=jnp.float32)
    out_ref[...] = jnp.maximum(o + blin, 0.0)


def kernel(x, edge_index, edge_weight, W1, b1, W2, b2, Wlin, blin):
    B = x.shape[0]
    zeros = jnp.zeros((_FLAT,), jnp.float32)
    m_parts, deg_parts = _sc_build_adj()(
        edge_index[0], edge_index[1], edge_weight, zeros)
    m_parts = m_parts.reshape(2, _N, _STRIDE)
    deg_parts = deg_parts.reshape(2, _DEGP)

    bm = 512
    grid = (B // bm,)
    out = pl.pallas_call(
        _tc_body,
        grid=grid,
        in_specs=[
            pl.BlockSpec((bm, _N), lambda i: (i, 0)),
            pl.BlockSpec((2, _N, _STRIDE), lambda i: (0, 0, 0)),
            pl.BlockSpec((2, _DEGP), lambda i: (0, 0)),
            pl.BlockSpec(memory_space=pltpu.SMEM),
            pl.BlockSpec(memory_space=pltpu.SMEM),
            pl.BlockSpec(memory_space=pltpu.SMEM),
            pl.BlockSpec(memory_space=pltpu.SMEM),
            pl.BlockSpec((_N, 1), lambda i: (0, 0)),
            pl.BlockSpec(memory_space=pltpu.SMEM),
        ],
        out_specs=pl.BlockSpec((bm, 1), lambda i: (i, 0)),
        out_shape=jax.ShapeDtypeStruct((B, 1), jnp.float32),
        scratch_shapes=[pltpu.VMEM((_N, _N), jnp.bfloat16)],
    )(
        x, m_parts, deg_parts,
        W1.reshape(1, 1), b1.reshape(1, 1),
        W2.reshape(1, 1), b2.reshape(1, 1),
        Wlin, blin.reshape(1, 1),
    )
    return out


# speedup vs baseline: 15.3293x; 1.0070x over previous
import functools

import jax
import jax.numpy as jnp
from jax import lax
from jax.experimental import pallas as pl
from jax.experimental.pallas import tpu as pltpu
from jax.experimental.pallas import tpu_sc as plsc

_N = 676
_E = 8192
_ROWS = 680
_PLANES = 6
_PLANE = _ROWS * 128
_FLAT = _PLANES * _PLANE
_CHUNK = _FLAT // 16
_EPW = _E // 32


def _sc_body(src_hbm, dst_hbm, w_hbm, zeros_hbm, m_out,
             srcv, dstv, wv, midx, zbuf, m_sp):
    c = lax.axis_index("c")
    s = lax.axis_index("s")
    wid = c * 16 + s
    base = wid * _EPW

    pltpu.sync_copy(zeros_hbm.at[pl.ds(0, _CHUNK)], zbuf)
    pltpu.sync_copy(zbuf, m_sp.at[pl.ds(s * _CHUNK, _CHUNK)])

    for j in range(2):
        off = base + j * 128
        pltpu.sync_copy(src_hbm.at[pl.ds(off, 128)], srcv.at[j])
        pltpu.sync_copy(dst_hbm.at[pl.ds(off, 128)], dstv.at[j])
        pltpu.sync_copy(w_hbm.at[pl.ds(off, 128)], wv.at[j])

    for j in range(2):
        for k in range(8):
            sl = pl.ds(k * 16, 16)
            sv = srcv[j, sl]
            dv = dstv[j, sl]
            midx[j, sl] = ((dv >> 7) * _PLANE + sv * 128 + (dv & 127))

    plsc.subcore_barrier()

    for j in range(2):
        pltpu.sync_copy(wv.at[j], m_sp.at[midx.at[j]], add=True)

    plsc.subcore_barrier()

    pltpu.sync_copy(m_sp.at[pl.ds(s * _CHUNK, _CHUNK)], zbuf)
    pltpu.sync_copy(zbuf, m_out.at[pl.ds(c * _FLAT + s * _CHUNK, _CHUNK)])


@functools.lru_cache(maxsize=1)
def _sc_scatter_adj():
    return pl.kernel(
        _sc_body,
        out_type=jax.ShapeDtypeStruct((2 * _FLAT,), jnp.float32),
        mesh=plsc.VectorSubcoreMesh(core_axis_name="c", subcore_axis_name="s"),
        scratch_types=[
            pltpu.VMEM((2, 128), jnp.int32),
            pltpu.VMEM((2, 128), jnp.int32),
            pltpu.VMEM((2, 128), jnp.float32),
            pltpu.VMEM((2, 128), jnp.int32),
            pltpu.VMEM((_CHUNK,), jnp.float32),
            pltpu.VMEM_SHARED((_FLAT,), jnp.float32),
        ],
    )


def _tc_body(x_ref, m_ref, w1_ref, b1_ref, w2_ref, b2_ref,
             wlin_ref, blin_ref, out_ref, ms_ref):
    @pl.when(pl.program_id(0) == 0)
    def _():
        planes = []
        degs = []
        for k in range(_PLANES):
            r = _ROWS * k
            mk = m_ref[r:r + _ROWS, :] + m_ref[_PLANES * _ROWS + r:
                                               _PLANES * _ROWS + r + _ROWS, :]
            planes.append(mk)
            degs.append(jnp.sum(mk, axis=0, keepdims=True))
        deg = jnp.concatenate(degs, axis=1) + 1.0
        nid = lax.broadcasted_iota(jnp.int32, (1, _PLANES * 128), 1)
        dinv = jnp.where(nid < _N, lax.rsqrt(deg), 0.0)
        dinv_col = jnp.transpose(dinv)[:_ROWS, :]
        rows = lax.broadcasted_iota(jnp.int32, (_ROWS, 128), 0)
        cols = lax.broadcasted_iota(jnp.int32, (_ROWS, 128), 1)
        for k in range(_PLANES):
            eye_k = jnp.where(rows == 128 * k + cols, 1.0, 0.0)
            msk = dinv_col * (planes[k] + eye_k) * dinv[:, 128 * k:
                                                        128 * (k + 1)]
            ms_ref[_ROWS * k:_ROWS * (k + 1), :] = msk.astype(jnp.bfloat16)

    w1 = w1_ref[0, 0]
    b1 = b1_ref[0, 0]
    w2 = w2_ref[0, 0]
    b2 = b2_ref[0, 0]
    blin = blin_ref[0, 0]

    def conv(h):
        hk = [jnp.dot(h, ms_ref[_ROWS * k:_ROWS * k + _N, :],
                      preferred_element_type=jnp.float32)
              for k in range(_PLANES)]
        return jnp.concatenate(hk, axis=1)

    h = x_ref[...].astype(jnp.bfloat16)
    h = jnp.maximum(conv(h) * w1 + b1, 0.0)[:, :_N].astype(jnp.bfloat16)
    h = jnp.maximum(conv(h) * w2 + b2, 0.0)[:, :_N]
    o = jnp.sum(h * wlin_ref[...], axis=1, keepdims=True)
    out_ref[...] = jnp.maximum(o + blin, 0.0)


def kernel(x, edge_index, edge_weight, W1, b1, W2, b2, Wlin, blin):
    B = x.shape[0]
    zeros = jnp.zeros((_CHUNK,), jnp.float32)
    m_flat = _sc_scatter_adj()(edge_index[0], edge_index[1], edge_weight,
                               zeros)
    m2d = m_flat.reshape(2 * _PLANES * _ROWS, 128)

    bm = 512
    grid = (B // bm,)
    out = pl.pallas_call(
        _tc_body,
        grid=grid,
        in_specs=[
            pl.BlockSpec((bm, _N), lambda i: (i, 0)),
            pl.BlockSpec((2 * _PLANES * _ROWS, 128), lambda i: (0, 0)),
            pl.BlockSpec(memory_space=pltpu.SMEM),
            pl.BlockSpec(memory_space=pltpu.SMEM),
            pl.BlockSpec(memory_space=pltpu.SMEM),
            pl.BlockSpec(memory_space=pltpu.SMEM),
            pl.BlockSpec((1, _N), lambda i: (0, 0)),
            pl.BlockSpec(memory_space=pltpu.SMEM),
        ],
        out_specs=pl.BlockSpec((bm, 1), lambda i: (i, 0)),
        out_shape=jax.ShapeDtypeStruct((B, 1), jnp.float32),
        scratch_shapes=[pltpu.VMEM((_PLANES * _ROWS, 128), jnp.bfloat16)],
    )(
        x, m2d,
        W1.reshape(1, 1), b1.reshape(1, 1),
        W2.reshape(1, 1), b2.reshape(1, 1),
        Wlin.reshape(1, _N), blin.reshape(1, 1),
    )
    return out


# speedup vs baseline: 18.3135x; 1.1947x over previous
import functools

import jax
import jax.numpy as jnp
from jax import lax
from jax.experimental import pallas as pl
from jax.experimental.pallas import tpu as pltpu
from jax.experimental.pallas import tpu_sc as plsc

_N = 676
_E = 8192
_ROWS = 680
_PLANES = 6
_PLANE = _ROWS * 128
_FLAT = _PLANES * _PLANE
_CHUNK = _FLAT // 16
_EPW = _E // 32


def _sc_body(src_hbm, dst_hbm, w_hbm, zeros_hbm, m_out,
             srcv, dstv, wv, midx, zbuf, m_sp):
    c = lax.axis_index("c")
    s = lax.axis_index("s")
    wid = c * 16 + s
    base = wid * _EPW

    pltpu.sync_copy(zeros_hbm.at[pl.ds(0, _CHUNK)], zbuf)
    pltpu.sync_copy(zbuf, m_sp.at[pl.ds(s * _CHUNK, _CHUNK)])

    for j in range(2):
        off = base + j * 128
        pltpu.sync_copy(src_hbm.at[pl.ds(off, 128)], srcv.at[j])
        pltpu.sync_copy(dst_hbm.at[pl.ds(off, 128)], dstv.at[j])
        pltpu.sync_copy(w_hbm.at[pl.ds(off, 128)], wv.at[j])

    for j in range(2):
        for k in range(8):
            sl = pl.ds(k * 16, 16)
            sv = srcv[j, sl]
            dv = dstv[j, sl]
            midx[j, sl] = ((dv >> 7) * _PLANE + sv * 128 + (dv & 127))

    plsc.subcore_barrier()

    for j in range(2):
        pltpu.sync_copy(wv.at[j], m_sp.at[midx.at[j]], add=True)

    plsc.subcore_barrier()

    pltpu.sync_copy(m_sp.at[pl.ds(s * _CHUNK, _CHUNK)], zbuf)
    pltpu.sync_copy(zbuf, m_out.at[pl.ds(c * _FLAT + s * _CHUNK, _CHUNK)])


@functools.lru_cache(maxsize=1)
def _sc_scatter_adj():
    return pl.kernel(
        _sc_body,
        out_type=jax.ShapeDtypeStruct((2 * _FLAT,), jnp.float32),
        mesh=plsc.VectorSubcoreMesh(core_axis_name="c", subcore_axis_name="s"),
        scratch_types=[
            pltpu.VMEM((2, 128), jnp.int32),
            pltpu.VMEM((2, 128), jnp.int32),
            pltpu.VMEM((2, 128), jnp.float32),
            pltpu.VMEM((2, 128), jnp.int32),
            pltpu.VMEM((_CHUNK,), jnp.float32),
            pltpu.VMEM_SHARED((_FLAT,), jnp.float32),
        ],
    )


def _tc_body(x_ref, m_ref, w1_ref, b1_ref, w2_ref, b2_ref,
             wlin_ref, blin_ref, out_ref, ms_ref):
    @pl.when(pl.program_id(0) == 0)
    def _():
        planes = []
        degs = []
        for k in range(_PLANES):
            r = _ROWS * k
            mk = m_ref[r:r + _ROWS, :] + m_ref[_PLANES * _ROWS + r:
                                               _PLANES * _ROWS + r + _ROWS, :]
            planes.append(mk)
            degs.append(jnp.sum(mk, axis=0, keepdims=True))
        deg = jnp.concatenate(degs, axis=1) + 1.0
        nid = lax.broadcasted_iota(jnp.int32, (1, _PLANES * 128), 1)
        dinv = jnp.where(nid < _N, lax.rsqrt(deg), 0.0)
        dinv_col = jnp.transpose(dinv)[:_ROWS, :]
        rows = lax.broadcasted_iota(jnp.int32, (_ROWS, 128), 0)
        cols = lax.broadcasted_iota(jnp.int32, (_ROWS, 128), 1)
        for k in range(_PLANES):
            eye_k = jnp.where(rows == 128 * k + cols, 1.0, 0.0)
            msk = dinv_col * (planes[k] + eye_k) * dinv[:, 128 * k:
                                                        128 * (k + 1)]
            ms_ref[:, 128 * k:128 * (k + 1)] = msk.astype(jnp.bfloat16)

    w1 = w1_ref[0, 0]
    b1 = b1_ref[0, 0]
    w2 = w2_ref[0, 0]
    b2 = b2_ref[0, 0]
    blin = blin_ref[0, 0]

    def conv(h):
        return jnp.dot(h, ms_ref[:_N, :],
                       preferred_element_type=jnp.float32)

    h = x_ref[...]
    h = jnp.maximum(conv(h) * w1 + b1, 0.0)[:, :_N].astype(jnp.bfloat16)
    h = jnp.maximum(conv(h) * w2 + b2, 0.0)[:, :_N]
    o = jnp.sum(h * wlin_ref[...], axis=1, keepdims=True)
    out_ref[...] = jnp.maximum(o + blin, 0.0)


def kernel(x, edge_index, edge_weight, W1, b1, W2, b2, Wlin, blin):
    B = x.shape[0]
    zeros = jnp.zeros((_CHUNK,), jnp.float32)
    m_flat = _sc_scatter_adj()(edge_index[0], edge_index[1], edge_weight,
                               zeros)
    m2d = m_flat.reshape(2 * _PLANES * _ROWS, 128)

    bm = 512
    grid = (B // bm,)
    out = pl.pallas_call(
        _tc_body,
        grid=grid,
        in_specs=[
            pl.BlockSpec((bm, _N), lambda i: (i, 0)),
            pl.BlockSpec((2 * _PLANES * _ROWS, 128), lambda i: (0, 0)),
            pl.BlockSpec(memory_space=pltpu.SMEM),
            pl.BlockSpec(memory_space=pltpu.SMEM),
            pl.BlockSpec(memory_space=pltpu.SMEM),
            pl.BlockSpec(memory_space=pltpu.SMEM),
            pl.BlockSpec((1, _N), lambda i: (0, 0)),
            pl.BlockSpec(memory_space=pltpu.SMEM),
        ],
        out_specs=pl.BlockSpec((bm, 1), lambda i: (i, 0)),
        out_shape=jax.ShapeDtypeStruct((B, 1), jnp.float32),
        scratch_shapes=[pltpu.VMEM((_ROWS, _PLANES * 128), jnp.bfloat16)],
    )(
        x.astype(jnp.bfloat16), m2d,
        W1.reshape(1, 1), b1.reshape(1, 1),
        W2.reshape(1, 1), b2.reshape(1, 1),
        Wlin.reshape(1, _N), blin.reshape(1, 1),
    )
    return out
